# baseline (speedup 1.0000x reference)
import jax
import jax.numpy as jnp
from jax import lax
from jax.experimental import pallas as pl
from jax.experimental.pallas import tpu as pltpu, tpu_sc as plsc

VOCAB = 100000
LENGTH = 200
EMBED_DIM = 64
BATCH = 4096

_TR = LENGTH // 8
_BC = BATCH // 128
_NCH = LENGTH // 2


def _body(ids_hbm, table_hbm, pos_hbm, out_hbm,
          idx_v, g_v, o_v, pos_v, gsem0, gsem1, osem0, osem1):
    w = lax.axis_index("c") * 16 + lax.axis_index("s")
    gsems = (gsem0, gsem1)
    osems = (osem0, osem1)

    pltpu.sync_copy(ids_hbm.at[:, pl.ds(w, 1), :, :], idx_v)
    pltpu.sync_copy(pos_hbm, pos_v)

    iota = jax.lax.iota(jnp.int32, 16)
    rowvecs = [[iota + (i * 128 + g * 16) for g in range(8)] for i in range(2)]

    def issue_gather(q, r):
        for i in range(2):
            l = 2 * q + i
            tr = l // 8
            lr = l % 8
            pltpu.async_copy(table_hbm.at[idx_v.at[tr, 0, lr]],
                             g_v.at[r, pl.ds(i * 128, 128)], gsems[r])

    def wait_gather(r):
        for i in range(2):
            pltpu.make_async_copy(table_hbm.at[idx_v.at[0, 0, 0]],
                                  g_v.at[r, pl.ds(i * 128, 128)],
                                  gsems[r]).wait()

    def compute(q, r):
        for i in range(2):
            l = 2 * q + i

            def k2loop(k2, carry):
                pv = pos_v[l, pl.ds(k2 * 16, 16)]
                for j in range(16):
                    d = k2 * 16 + j
                    psv = pv.at[jnp.full((16,), j)].get(
                        mode="promise_in_bounds")
                    dv = jnp.full((16,), d)
                    k = 2 * k2 + j // 8
                    for g in range(8):
                        v = plsc.load_gather(g_v.at[r], [rowvecs[i][g], dv])
                        o_v[r, i, k, 0, j % 8, pl.ds(g * 16, 16)] = v + psv
                return carry

            lax.fori_loop(0, EMBED_DIM // 16, k2loop, 0)

    def issue_out(q, r):
        pltpu.async_copy(o_v.at[r],
                         out_hbm.at[pl.ds(2 * q, 2), :, pl.ds(w, 1), :, :],
                         osems[r])

    def wait_out(r):
        pltpu.make_async_copy(o_v.at[r],
                              out_hbm.at[pl.ds(0, 2), :, pl.ds(0, 1), :, :],
                              osems[r]).wait()

    issue_gather(0, 0)
    issue_gather(1, 1)
    for r in range(2):
        wait_gather(r)
        compute(r, r)
        issue_gather(r + 2, r)
        issue_out(r, r)

    def pair_body(t, carry):
        for r in range(2):
            q = 2 * t + r
            wait_gather(r)
            wait_out(r)
            compute(q, r)

            @pl.when(q + 2 < _NCH)
            def _():
                issue_gather(q + 2, r)

            issue_out(q, r)
        return carry

    lax.fori_loop(1, _NCH // 2, pair_body, 0)
    wait_out(0)
    wait_out(1)


@jax.jit
def _run(ids4, token_table, position_table):
    mesh = plsc.VectorSubcoreMesh(core_axis_name="c", subcore_axis_name="s")
    return pl.kernel(
        _body,
        out_type=jax.ShapeDtypeStruct((LENGTH, 8, _BC, 8, 128), jnp.float32),
        mesh=mesh,
        scratch_types=[
            pltpu.VMEM((_TR, 1, 8, 128), jnp.int32),
            pltpu.VMEM((2, 256, EMBED_DIM), jnp.float32),
            pltpu.VMEM((2, 2, 8, 1, 8, 128), jnp.float32),
            pltpu.VMEM((LENGTH, EMBED_DIM), jnp.float32),
            pltpu.SemaphoreType.DMA,
            pltpu.SemaphoreType.DMA,
            pltpu.SemaphoreType.DMA,
            pltpu.SemaphoreType.DMA,
        ],
        compiler_params=pltpu.CompilerParams(use_tc_tiling_on_sc=False,
                                             needs_layout_passes=False,
                                             disable_bounds_checks=True),
    )(ids4, token_table, position_table)


def kernel(input_ids, token_table, position_table):
    ids4 = (input_ids.astype(jnp.int32).T
            .reshape(_TR, 8, _BC, 128).transpose(0, 2, 1, 3))
    out5 = _run(ids4, token_table, position_table)
    return out5.transpose(2, 4, 0, 1, 3).reshape(BATCH, LENGTH, EMBED_DIM)


# speedup vs baseline: 2.3122x; 2.3122x over previous
import jax
import jax.numpy as jnp
from jax import lax
from jax.experimental import pallas as pl
from jax.experimental.pallas import tpu as pltpu, tpu_sc as plsc

VOCAB = 100000
LENGTH = 200
EMBED_DIM = 64
BATCH = 4096

_TR = LENGTH // 8
_BC = BATCH // 128
_NCH = LENGTH // 2
_OP = 129


def _body(ids_hbm, table_hbm, pos_hbm, out_hbm,
          idx_v, g_v, o_v, pos_v, gsem0, gsem1, osem0, osem1):
    w = lax.axis_index("c") * 16 + lax.axis_index("s")
    gsems = (gsem0, gsem1)
    osems = (osem0, osem1)

    pltpu.sync_copy(ids_hbm.at[:, pl.ds(w, 1), :, :], idx_v)
    pltpu.sync_copy(pos_hbm, pos_v)

    iota = jax.lax.iota(jnp.int32, 16)
    kvecs = [(iota + 16 * k2) // 8 for k2 in range(4)]
    ddvecs = [(iota + 16 * k2) % 8 for k2 in range(4)]
    zvec = jnp.zeros((16,), jnp.int32)

    def issue_gather(q, r):
        for i in range(2):
            l = 2 * q + i
            tr = l // 8
            lr = l % 8
            pltpu.async_copy(table_hbm.at[idx_v.at[tr, 0, lr]],
                             g_v.at[r, pl.ds(i * 128, 128)], gsems[r])

    def wait_gather(r):
        for i in range(2):
            pltpu.make_async_copy(table_hbm.at[idx_v.at[0, 0, 0]],
                                  g_v.at[r, pl.ds(i * 128, 128)],
                                  gsems[r]).wait()

    def compute(q, r):
        for i in range(2):
            l = 2 * q + i
            pvs = [pos_v[l, pl.ds(16 * k2, 16)] for k2 in range(4)]

            def bbloop(bb, carry):
                bbs = jnp.full((16,), bb)
                for k2 in range(4):
                    v = g_v[r, i * 128 + bb, pl.ds(16 * k2, 16)] + pvs[k2]
                    plsc.store_scatter(o_v.at[r, i],
                                       [kvecs[k2], zvec, ddvecs[k2], bbs], v)
                return carry

            lax.fori_loop(0, 128, bbloop, 0)

    def issue_out(q, r):
        pltpu.async_copy(o_v.at[r, :, :, :, :, pl.ds(0, 128)],
                         out_hbm.at[pl.ds(2 * q, 2), :, pl.ds(w, 1), :, :],
                         osems[r])

    def wait_out(r):
        pltpu.make_async_copy(o_v.at[r, :, :, :, :, pl.ds(0, 128)],
                              out_hbm.at[pl.ds(0, 2), :, pl.ds(0, 1), :, :],
                              osems[r]).wait()

    issue_gather(0, 0)
    issue_gather(1, 1)
    for r in range(2):
        wait_gather(r)
        compute(r, r)
        issue_gather(r + 2, r)
        issue_out(r, r)

    def pair_body(t, carry):
        for r in range(2):
            q = 2 * t + r
            wait_gather(r)
            wait_out(r)
            compute(q, r)

            @pl.when(q + 2 < _NCH)
            def _():
                issue_gather(q + 2, r)

            issue_out(q, r)
        return carry

    lax.fori_loop(1, _NCH // 2, pair_body, 0)
    wait_out(0)
    wait_out(1)


@jax.jit
def _run(ids4, token_table, position_table):
    mesh = plsc.VectorSubcoreMesh(core_axis_name="c", subcore_axis_name="s")
    return pl.kernel(
        _body,
        out_type=jax.ShapeDtypeStruct((LENGTH, 8, _BC, 8, 128), jnp.float32),
        mesh=mesh,
        scratch_types=[
            pltpu.VMEM((_TR, 1, 8, 128), jnp.int32),
            pltpu.VMEM((2, 256, EMBED_DIM), jnp.float32),
            pltpu.VMEM((2, 2, 8, 1, 8, _OP), jnp.float32),
            pltpu.VMEM((LENGTH, EMBED_DIM), jnp.float32),
            pltpu.SemaphoreType.DMA,
            pltpu.SemaphoreType.DMA,
            pltpu.SemaphoreType.DMA,
            pltpu.SemaphoreType.DMA,
        ],
        compiler_params=pltpu.CompilerParams(use_tc_tiling_on_sc=False,
                                             needs_layout_passes=False,
                                             disable_bounds_checks=True),
    )(ids4, token_table, position_table)


def kernel(input_ids, token_table, position_table):
    ids4 = (input_ids.astype(jnp.int32).T
            .reshape(_TR, 8, _BC, 128).transpose(0, 2, 1, 3))
    out5 = _run(ids4, token_table, position_table)
    return out5.transpose(2, 4, 0, 1, 3).reshape(BATCH, LENGTH, EMBED_DIM)


# speedup vs baseline: 2.3946x; 1.0357x over previous
import jax
import jax.numpy as jnp
from jax import lax
from jax.experimental import pallas as pl
from jax.experimental.pallas import tpu as pltpu, tpu_sc as plsc

VOCAB = 100000
LENGTH = 200
EMBED_DIM = 64
BATCH = 4096

_TR = LENGTH // 8
_BC = BATCH // 128
_NCH = LENGTH // 2
_OP = 129


def _body(ids_hbm, table_hbm, pos_hbm, out_hbm,
          idx_v, g_v, o_v, pos_v, gsem0, gsem1, osem0, osem1):
    w = lax.axis_index("c") * 16 + lax.axis_index("s")
    gsems = (gsem0, gsem1)
    osems = (osem0, osem1)

    pltpu.sync_copy(ids_hbm.at[:, pl.ds(w, 1), :, :], idx_v)
    pltpu.sync_copy(pos_hbm, pos_v)

    iota = jax.lax.iota(jnp.int32, 16)
    kvecs = [(iota + 16 * k2) // 8 for k2 in range(4)]
    ddvecs = [(iota + 16 * k2) % 8 for k2 in range(4)]
    zvec = jnp.zeros((16,), jnp.int32)

    def issue_gather(q, r):
        for i in range(2):
            l = 2 * q + i
            tr = l // 8
            lr = l % 8
            pltpu.async_copy(table_hbm.at[idx_v.at[tr, 0, lr]],
                             g_v.at[r, pl.ds(i * 128, 128)], gsems[r])

    def wait_gather(r):
        for i in range(2):
            pltpu.make_async_copy(table_hbm.at[idx_v.at[0, 0, 0]],
                                  g_v.at[r, pl.ds(i * 128, 128)],
                                  gsems[r]).wait()

    def compute(q, r):
        for i in range(2):
            l = 2 * q + i
            pvs = [pos_v[l, pl.ds(16 * k2, 16)] for k2 in range(4)]

            def bbloop(bb, carry):
                bbs = jnp.full((16,), bb)
                for k2 in range(4):
                    v = g_v[r, i * 128 + bb, pl.ds(16 * k2, 16)] + pvs[k2]
                    plsc.store_scatter(o_v.at[r, i],
                                       [kvecs[k2], zvec, ddvecs[k2], bbs], v)
                return carry

            lax.fori_loop(0, 128, bbloop, 0, unroll=4)

    def issue_out(q, r):
        pltpu.async_copy(o_v.at[r, :, :, :, :, pl.ds(0, 128)],
                         out_hbm.at[pl.ds(2 * q, 2), :, pl.ds(w, 1), :, :],
                         osems[r])

    def wait_out(r):
        pltpu.make_async_copy(o_v.at[r, :, :, :, :, pl.ds(0, 128)],
                              out_hbm.at[pl.ds(0, 2), :, pl.ds(0, 1), :, :],
                              osems[r]).wait()

    issue_gather(0, 0)
    issue_gather(1, 1)
    for r in range(2):
        wait_gather(r)
        compute(r, r)
        issue_gather(r + 2, r)
        issue_out(r, r)

    def pair_body(t, carry):
        for r in range(2):
            q = 2 * t + r
            wait_gather(r)
            wait_out(r)
            compute(q, r)

            @pl.when(q + 2 < _NCH)
            def _():
                issue_gather(q + 2, r)

            issue_out(q, r)
        return carry

    lax.fori_loop(1, _NCH // 2, pair_body, 0)
    wait_out(0)
    wait_out(1)


@jax.jit
def _run(ids4, token_table, position_table):
    mesh = plsc.VectorSubcoreMesh(core_axis_name="c", subcore_axis_name="s")
    return pl.kernel(
        _body,
        out_type=jax.ShapeDtypeStruct((LENGTH, 8, _BC, 8, 128), jnp.float32),
        mesh=mesh,
        scratch_types=[
            pltpu.VMEM((_TR, 1, 8, 128), jnp.int32),
            pltpu.VMEM((2, 256, EMBED_DIM), jnp.float32),
            pltpu.VMEM((2, 2, 8, 1, 8, _OP), jnp.float32),
            pltpu.VMEM((LENGTH, EMBED_DIM), jnp.float32),
            pltpu.SemaphoreType.DMA,
            pltpu.SemaphoreType.DMA,
            pltpu.SemaphoreType.DMA,
            pltpu.SemaphoreType.DMA,
        ],
        compiler_params=pltpu.CompilerParams(use_tc_tiling_on_sc=False,
                                             needs_layout_passes=False,
                                             disable_bounds_checks=True),
    )(ids4, token_table, position_table)


def kernel(input_ids, token_table, position_table):
    ids4 = (input_ids.astype(jnp.int32).T
            .reshape(_TR, 8, _BC, 128).transpose(0, 2, 1, 3))
    out5 = _run(ids4, token_table, position_table)
    return out5.transpose(2, 4, 0, 1, 3).reshape(BATCH, LENGTH, EMBED_DIM)


# speedup vs baseline: 6.3603x; 2.6561x over previous
import jax
import jax.numpy as jnp
from jax import lax
from jax.experimental import pallas as pl
from jax.experimental.pallas import tpu as pltpu, tpu_sc as plsc

VOCAB = 100000
LENGTH = 200
EMBED_DIM = 64
BATCH = 4096

_TR = LENGTH // 8
_BC = BATCH // 128
_NCH = LENGTH // 2
_OP = 129


def _body(ids_hbm, table_hbm, pos_hbm, out_hbm,
          idx_v, g_v, o_v, pos_v, gsem0, gsem1, osem0, osem1):
    w = lax.axis_index("c") * 16 + lax.axis_index("s")
    gsems = (gsem0, gsem1)
    osems = (osem0, osem1)

    pltpu.sync_copy(ids_hbm.at[:, pl.ds(w, 1), :, :], idx_v)
    pltpu.sync_copy(pos_hbm, pos_v)

    iota = jax.lax.iota(jnp.int32, 16)
    kvecs = [(iota + 16 * k2) // 8 for k2 in range(4)]
    ddvecs = [(iota + 16 * k2) % 8 for k2 in range(4)]
    zvec = jnp.zeros((16,), jnp.int32)

    def issue_gather(q, r):
        for i in range(2):
            l = 2 * q + i
            tr = l // 8
            lr = l % 8
            pltpu.async_copy(table_hbm.at[idx_v.at[tr, 0, lr]],
                             g_v.at[r, pl.ds(i * 128, 128)], gsems[r])

    def wait_gather(r):
        for i in range(2):
            pltpu.make_async_copy(table_hbm.at[idx_v.at[0, 0, 0]],
                                  g_v.at[r, pl.ds(i * 128, 128)],
                                  gsems[r]).wait()

    def compute(q, r):
        for i in range(2):
            l = 2 * q + i
            pvs = [pos_v[l, pl.ds(16 * k2, 16)] for k2 in range(4)]

            @plsc.parallel_loop(0, 128, step=1, unroll=4)
            def _bbloop(bb):
                bbs = jnp.full((16,), bb)
                for k2 in range(4):
                    v = g_v[r, i * 128 + bb, pl.ds(16 * k2, 16)] + pvs[k2]
                    plsc.store_scatter(o_v.at[r, i],
                                       [kvecs[k2], zvec, ddvecs[k2], bbs], v)

    def issue_out(q, r):
        pltpu.async_copy(o_v.at[r, :, :, :, :, pl.ds(0, 128)],
                         out_hbm.at[pl.ds(2 * q, 2), :, pl.ds(w, 1), :, :],
                         osems[r])

    def wait_out(r):
        pltpu.make_async_copy(o_v.at[r, :, :, :, :, pl.ds(0, 128)],
                              out_hbm.at[pl.ds(0, 2), :, pl.ds(0, 1), :, :],
                              osems[r]).wait()

    issue_gather(0, 0)
    issue_gather(1, 1)
    for r in range(2):
        wait_gather(r)
        compute(r, r)
        issue_gather(r + 2, r)
        issue_out(r, r)

    def pair_body(t, carry):
        for r in range(2):
            q = 2 * t + r
            wait_gather(r)
            wait_out(r)
            compute(q, r)

            @pl.when(q + 2 < _NCH)
            def _():
                issue_gather(q + 2, r)

            issue_out(q, r)
        return carry

    lax.fori_loop(1, _NCH // 2, pair_body, 0)
    wait_out(0)
    wait_out(1)


@jax.jit
def _run(ids4, token_table, position_table):
    mesh = plsc.VectorSubcoreMesh(core_axis_name="c", subcore_axis_name="s")
    return pl.kernel(
        _body,
        out_type=jax.ShapeDtypeStruct((LENGTH, 8, _BC, 8, 128), jnp.float32),
        mesh=mesh,
        scratch_types=[
            pltpu.VMEM((_TR, 1, 8, 128), jnp.int32),
            pltpu.VMEM((2, 256, EMBED_DIM), jnp.float32),
            pltpu.VMEM((2, 2, 8, 1, 8, _OP), jnp.float32),
            pltpu.VMEM((LENGTH, EMBED_DIM), jnp.float32),
            pltpu.SemaphoreType.DMA,
            pltpu.SemaphoreType.DMA,
            pltpu.SemaphoreType.DMA,
            pltpu.SemaphoreType.DMA,
        ],
        compiler_params=pltpu.CompilerParams(use_tc_tiling_on_sc=False,
                                             needs_layout_passes=False,
                                             disable_bounds_checks=True),
    )(ids4, token_table, position_table)


def kernel(input_ids, token_table, position_table):
    ids4 = (input_ids.astype(jnp.int32).T
            .reshape(_TR, 8, _BC, 128).transpose(0, 2, 1, 3))
    out5 = _run(ids4, token_table, position_table)
    return out5.transpose(2, 4, 0, 1, 3).reshape(BATCH, LENGTH, EMBED_DIM)
